# baseline (speedup 1.0000x reference)
import jax
import jax.numpy as jnp
from jax.experimental import pallas as pl

VOCAB = 1000
VP = 1024
LP = 56
BB = 64


def _one_hot_body(x_ref, o_ref):
    idx = x_ref[...]
    iota = jax.lax.broadcasted_iota(jnp.int32, (BB, LP, VP), 2)
    idxp = jnp.pad(idx, ((0, 0), (0, LP - idx.shape[1])), constant_values=-1)
    o_ref[...] = (iota == idxp[:, :, None]).astype(jnp.float32)


def kernel(x):
    B, L = x.shape
    return pl.pallas_call(
        _one_hot_body,
        grid=(B // BB,),
        in_specs=[pl.BlockSpec((BB, L), lambda i: (i, 0))],
        out_specs=pl.BlockSpec((BB, LP, VP), lambda i: (i, 0, 0)),
        out_shape=jax.ShapeDtypeStruct((B, L, VOCAB), jnp.float32),
    )(x)


# speedup vs baseline: 1.2458x; 1.2458x over previous
import jax
import jax.numpy as jnp
from jax.experimental import pallas as pl

VOCAB = 1000
VP = 1024
LP = 56
BB = 64


def _one_hot_body(x_ref, o_ref):
    idx = x_ref[...]
    iota = jax.lax.broadcasted_iota(jnp.int32, (BB, LP, VP), 2)
    o_ref[...] = (iota == idx[:, :, None]).astype(jnp.float32)


def kernel(x):
    B, L = x.shape
    xp = jnp.pad(x, ((0, 0), (0, LP - L)), constant_values=-1)
    aligned = pl.pallas_call(
        _one_hot_body,
        grid=(B // BB,),
        in_specs=[pl.BlockSpec((BB, LP), lambda i: (i, 0))],
        out_specs=pl.BlockSpec((BB, LP, VP), lambda i: (i, 0, 0)),
        out_shape=jax.ShapeDtypeStruct((B, LP, VP), jnp.float32),
    )(xp)
    return aligned[:, :L, :VOCAB]
